# baseline (speedup 1.0000x reference)
import jax
import jax.numpy as jnp
from jax.experimental import pallas as pl
from jax.experimental.pallas import tpu as pltpu

_FILL = -10000.0
_V = 100000
_T = 2048
_H = 2048
_R = 65
_RP = 128
_VB = 2048
_NV = (_V + _VB - 1) // _VB


def _body(tok_ref, hid_ref, wt_ref, out_ref, rest_ref):
    v = pl.program_id(0)

    @pl.when(v == 0)
    def _():
        rest_ref[...] = jnp.dot(
            hid_ref[...], wt_ref[...], preferred_element_type=jnp.float32
        )

    base = v * _VB
    toks = tok_ref[...]
    has = jnp.any((toks >= base) & (toks < base + _VB))

    @pl.when(jnp.logical_not(has))
    def _():
        out_ref[...] = jnp.full((_T, _VB), _FILL, jnp.float32)

    @pl.when(has)
    def _():
        cols = jax.lax.broadcasted_iota(jnp.int32, (_RP, _VB), 1) + base
        ohb = tok_ref[:, 0:1] == cols
        mm = jnp.dot(
            rest_ref[...], ohb.astype(jnp.float32),
            preferred_element_type=jnp.float32,
        )
        out_ref[...] = jnp.where(jnp.any(ohb, axis=0)[None, :], mm, _FILL)


def kernel(hidden_states, W, token_ids):
    hid = hidden_states.reshape(_T, _H)
    wt = jnp.zeros((_H, _RP), jnp.float32).at[:, :_R].set(W.T)
    tok = jnp.broadcast_to(
        jnp.full((_RP,), -1, jnp.int32).at[:_R].set(token_ids)[:, None],
        (_RP, 128),
    )
    out = pl.pallas_call(
        _body,
        grid=(_NV,),
        in_specs=[
            pl.BlockSpec((_RP, 128), lambda v: (0, 0)),
            pl.BlockSpec((_T, _H), lambda v: (0, 0)),
            pl.BlockSpec((_H, _RP), lambda v: (0, 0)),
        ],
        out_specs=pl.BlockSpec((_T, _VB), lambda v: (0, v)),
        out_shape=jax.ShapeDtypeStruct((_T, _V), jnp.float32),
        scratch_shapes=[pltpu.VMEM((_T, _RP), jnp.float32)],
        compiler_params=pltpu.CompilerParams(
            dimension_semantics=("arbitrary",),
        ),
    )(tok, hid, wt)
    return out.reshape(1, _T, _V)


# speedup vs baseline: 1.0021x; 1.0021x over previous
import jax
import jax.numpy as jnp
from jax.experimental import pallas as pl
from jax.experimental.pallas import tpu as pltpu

_FILL = -10000.0
_V = 100000
_T = 2048
_H = 2048
_R = 65
_RP = 128
_VB = 2048
_NV = (_V + _VB - 1) // _VB


def _body(tok_ref, hid_ref, wt_ref, out_ref, rest_ref):
    v = pl.program_id(0)

    @pl.when(v == 0)
    def _():
        rest_ref[...] = jnp.dot(
            hid_ref[...], wt_ref[...], preferred_element_type=jnp.float32
        )

    base = v * _VB
    toks = tok_ref[...]
    has = jnp.any((toks >= base) & (toks < base + _VB))
    prev = base - 2 * _VB
    dirtied = (v >= 2) & jnp.any((toks >= prev) & (toks < prev + _VB))
    need_fill = jnp.logical_and(
        jnp.logical_not(has), (v < 2) | dirtied
    )

    @pl.when(need_fill)
    def _():
        out_ref[...] = jnp.full((_T, _VB), _FILL, jnp.float32)

    @pl.when(has)
    def _():
        cols = jax.lax.broadcasted_iota(jnp.int32, (_RP, _VB), 1) + base
        ohb = tok_ref[:, 0:1] == cols
        mm = jnp.dot(
            rest_ref[...], ohb.astype(jnp.float32),
            preferred_element_type=jnp.float32,
        )
        out_ref[...] = jnp.where(jnp.any(ohb, axis=0)[None, :], mm, _FILL)


def kernel(hidden_states, W, token_ids):
    hid = hidden_states.reshape(_T, _H)
    wt = jnp.zeros((_H, _RP), jnp.float32).at[:, :_R].set(W.T)
    tok = jnp.broadcast_to(
        jnp.full((_RP,), -1, jnp.int32).at[:_R].set(token_ids)[:, None],
        (_RP, 128),
    )
    out = pl.pallas_call(
        _body,
        grid=(_NV,),
        in_specs=[
            pl.BlockSpec((_RP, 128), lambda v: (0, 0)),
            pl.BlockSpec((_T, _H), lambda v: (0, 0)),
            pl.BlockSpec((_H, _RP), lambda v: (0, 0)),
        ],
        out_specs=pl.BlockSpec((_T, _VB), lambda v: (0, v)),
        out_shape=jax.ShapeDtypeStruct((_T, _V), jnp.float32),
        scratch_shapes=[pltpu.VMEM((_T, _RP), jnp.float32)],
        compiler_params=pltpu.CompilerParams(
            dimension_semantics=("arbitrary",),
        ),
    )(tok, hid, wt)
    return out.reshape(1, _T, _V)
